# baseline (speedup 1.0000x reference)
import functools

import jax
import jax.numpy as jnp
from jax import lax
from jax.experimental import pallas as pl
from jax.experimental.pallas import tpu as pltpu
from jax.experimental.pallas import tpu_sc as plsc

B, T, D = 4, 8192, 768
PAD = 100000
NC, NS = 2, 16
NW = NC * NS
PW = T // NW
C = 32
NCH = PW // C
KV = D // 16
NIT = NCH * B
NB = NIT // 2

_DN = lax.GatherDimensionNumbers(
    offset_dims=(), collapsed_slice_dims=(0,), start_index_map=(0,))

_mesh = plsc.VectorSubcoreMesh(core_axis_name="c", subcore_axis_name="s")


@functools.partial(
    pl.kernel,
    out_type=jax.ShapeDtypeStruct((B * T, D), jnp.float32),
    mesh=_mesh,
    scratch_types=[
        pltpu.VMEM((B, PW), jnp.int32),
        pltpu.VMEM((2, C), jnp.int32),
        pltpu.VMEM((2, C), jnp.float32),
        pltpu.VMEM((C, D), jnp.float32),
        pltpu.VMEM((2, C, D), jnp.float32),
        pltpu.SemaphoreType.DMA,
        pltpu.SemaphoreType.DMA((2,)),
        pltpu.SemaphoreType.DMA((2,)),
        pltpu.SemaphoreType.DMA,
    ],
)
def _emb_lookup(x_hbm, emb_hbm, pos_hbm, out_hbm,
                idxall, idxs2, mask2, pbuf, ebuf, isem, gsem, osem, psem):
    wid = lax.axis_index("s") * NC + lax.axis_index("c")
    pos_base = wid * PW

    def flat0_of(it):
        return (it % B) * T + pos_base + (it // B) * C

    def prep(it, slot):
        b, pc = it % B, it // B
        padv = jnp.zeros((16,), jnp.int32)
        for k in range(C // 16):
            sl = pl.ds(k * 16, 16)
            v = idxall[b, pl.ds(pc * C + k * 16, 16)]
            ispad = v == PAD
            idxs2[slot, sl] = jnp.where(ispad, 0, v)
            mask2[slot, sl] = jnp.where(ispad, 1.0, 0.0)
            padv = padv | jnp.where(ispad, 1, 0)
        lanes = lax.iota(jnp.int32, 16)
        for sh in (8, 4, 2, 1):
            perm = ((lanes + sh) & 15)[:, None]
            padv = padv | lax.gather(
                padv, perm, _DN, (1,),
                mode=lax.GatherScatterMode.PROMISE_IN_BOUNDS)
        return padv[0]

    def start_gather(slot):
        pltpu.async_copy(emb_hbm.at[idxs2.at[slot]], ebuf.at[slot],
                         gsem.at[slot])

    def wait_gather(slot):
        pltpu.make_async_copy(emb_hbm.at[idxs2.at[slot]], ebuf.at[slot],
                              gsem.at[slot]).wait()

    def start_pos(pc):
        pltpu.async_copy(pos_hbm.at[pl.ds(pos_base + pc * C, C)], pbuf, psem)

    def wait_pos(pc):
        pltpu.make_async_copy(pos_hbm.at[pl.ds(pos_base + pc * C, C)],
                              pbuf, psem).wait()

    def start_out(it, slot):
        pltpu.async_copy(ebuf.at[slot], out_hbm.at[pl.ds(flat0_of(it), C)],
                         osem.at[slot])

    def wait_out(it, slot):
        pltpu.make_async_copy(ebuf.at[slot],
                              out_hbm.at[pl.ds(flat0_of(it), C)],
                              osem.at[slot]).wait()

    def compute(slot, anypad):
        @plsc.parallel_loop(0, C, 1)
        def crow(r):
            for k in range(KV):
                sl = pl.ds(k * 16, 16)
                plsc.addupdate(ebuf.at[slot, r, sl], pbuf[r, sl])

        @pl.when(anypad != 0)
        def _():
            def rrow(r, c2):
                g16 = pl.multiple_of((r // 16) * 16, 16)
                mv = mask2[slot, pl.ds(g16, 16)]
                m = lax.gather(
                    mv, jnp.full((16, 1), r % 16, jnp.int32), _DN, (1,),
                    mode=lax.GatherScatterMode.PROMISE_IN_BOUNDS)
                km = 1.0 - m

                def rk(k, c3):
                    sl = pl.ds(pl.multiple_of(k * 16, 16), 16)
                    ebuf[slot, r, sl] = (km * ebuf[slot, r, sl]
                                         + m * pbuf[r, sl])
                    return c3

                lax.fori_loop(0, KV, rk, 0)
                return c2

            lax.fori_loop(0, C, rrow, 0)

    for b in range(B):
        pltpu.async_copy(x_hbm.at[pl.ds(b * T + pos_base, PW)],
                         idxall.at[b], isem)
    for b in range(B):
        pltpu.make_async_copy(x_hbm.at[pl.ds(b * T + pos_base, PW)],
                              idxall.at[b], isem).wait()
    start_pos(0)
    apad0 = prep(0, 0)
    start_gather(0)

    def body(i, anypad_e):
        e = 2 * i
        o = e + 1
        pc = i // 2

        @pl.when(i > 0)
        def _():
            wait_out(o - 2, 1)

        anypad_o = prep(o, 1)
        start_gather(1)

        @pl.when(i % 2 == 0)
        def _():
            wait_pos(pc)

        wait_gather(0)
        compute(0, anypad_e)
        start_out(e, 0)

        wait_gather(1)
        compute(1, anypad_o)
        start_out(o, 1)

        @pl.when((i % 2 == 1) & (pc + 1 < NCH))
        def _():
            start_pos(pc + 1)

        wait_out(e, 0)
        anypad_e2 = prep(jnp.minimum(e + 2, NIT - 1), 0)

        @pl.when(i < NB - 1)
        def _():
            start_gather(0)

        return anypad_e2

    lax.fori_loop(0, NB, body, apad0)

    wait_out(NIT - 1, 1)


def kernel(x, emb_table, pos_table):
    out = _emb_lookup(x.reshape(-1).astype(jnp.int32), emb_table, pos_table)
    return out.reshape(B, T, D)


# speedup vs baseline: 1.1189x; 1.1189x over previous
import functools

import jax
import jax.numpy as jnp
from jax import lax
from jax.experimental import pallas as pl
from jax.experimental.pallas import tpu as pltpu
from jax.experimental.pallas import tpu_sc as plsc

B, T, D = 4, 8192, 768
PAD = 100000
NC, NS = 2, 16
NW = NC * NS
PW = T // NW
C = 32
NCH = PW // C
KV = D // 16
NIT = NCH * B

_DN = lax.GatherDimensionNumbers(
    offset_dims=(), collapsed_slice_dims=(0,), start_index_map=(0,))

_mesh = plsc.VectorSubcoreMesh(core_axis_name="c", subcore_axis_name="s")


@functools.partial(
    pl.kernel,
    out_type=jax.ShapeDtypeStruct((B * T, D), jnp.float32),
    mesh=_mesh,
    scratch_types=[
        pltpu.VMEM((B, PW), jnp.int32),
        pltpu.VMEM((4, C), jnp.int32),
        pltpu.VMEM((4, C), jnp.float32),
        pltpu.VMEM((C, D), jnp.float32),
        pltpu.VMEM((4, C, D), jnp.float32),
        pltpu.SemaphoreType.DMA,
        pltpu.SemaphoreType.DMA((4,)),
        pltpu.SemaphoreType.DMA((4,)),
        pltpu.SemaphoreType.DMA,
    ],
)
def _emb_lookup(x_hbm, emb_hbm, pos_hbm, out_hbm,
                idxall, idxs4, mask4, pbuf, ebuf, isem, gsem, osem, psem):
    wid = lax.axis_index("s") * NC + lax.axis_index("c")
    pos_base = wid * PW

    def prep(it, slot):
        b, pc = it % B, it // B
        padv = jnp.zeros((16,), jnp.int32)
        for k in range(C // 16):
            sl = pl.ds(k * 16, 16)
            v = idxall[b, pl.ds(pc * C + k * 16, 16)]
            ispad = v == PAD
            idxs4[slot, sl] = jnp.where(ispad, 0, v)
            mask4[slot, sl] = jnp.where(ispad, 1.0, 0.0)
            padv = padv | jnp.where(ispad, 1, 0)
        lanes = lax.iota(jnp.int32, 16)
        for sh in (8, 4, 2, 1):
            perm = ((lanes + sh) & 15)[:, None]
            padv = padv | lax.gather(
                padv, perm, _DN, (1,),
                mode=lax.GatherScatterMode.PROMISE_IN_BOUNDS)
        return padv[0]

    def start_gather(slot):
        pltpu.async_copy(emb_hbm.at[idxs4.at[slot]], ebuf.at[slot],
                         gsem.at[slot])

    def wait_gather(slot):
        pltpu.make_async_copy(emb_hbm.at[idxs4.at[slot]], ebuf.at[slot],
                              gsem.at[slot]).wait()

    def start_pos(pc):
        pltpu.async_copy(pos_hbm.at[pl.ds(pos_base + pc * C, C)], pbuf, psem)

    def wait_pos():
        pltpu.make_async_copy(pos_hbm.at[pl.ds(0, C)], pbuf, psem).wait()

    def start_out(it, slot):
        flat0 = (it % B) * T + pos_base + (it // B) * C
        pltpu.async_copy(ebuf.at[slot], out_hbm.at[pl.ds(flat0, C)],
                         osem.at[slot])

    def wait_out(slot):
        pltpu.make_async_copy(ebuf.at[slot], out_hbm.at[pl.ds(0, C)],
                              osem.at[slot]).wait()

    def rare_fix(slot, anypad):
        @pl.when(anypad != 0)
        def _():
            def rrow(r, c2):
                g16 = pl.multiple_of((r // 16) * 16, 16)
                mv = mask4[slot, pl.ds(g16, 16)]
                m = lax.gather(
                    mv, jnp.full((16, 1), r % 16, jnp.int32), _DN, (1,),
                    mode=lax.GatherScatterMode.PROMISE_IN_BOUNDS)
                km = 1.0 - m

                def rk(k, c3):
                    sl = pl.ds(pl.multiple_of(k * 16, 16), 16)
                    ebuf[slot, r, sl] = (km * ebuf[slot, r, sl]
                                         + m * pbuf[r, sl])
                    return c3

                lax.fori_loop(0, KV, rk, 0)
                return c2

            lax.fori_loop(0, C, rrow, 0)

    def fused_compute(slotA, slotB, apA, apB):
        @plsc.parallel_loop(0, C, 1)
        def crow(r):
            for k in range(KV):
                sl = pl.ds(k * 16, 16)
                v = pbuf[r, sl]
                plsc.addupdate(ebuf.at[slotA, r, sl], v)
                plsc.addupdate(ebuf.at[slotB, r, sl], v)

        rare_fix(slotA, apA)
        rare_fix(slotB, apB)

    for b in range(B):
        pltpu.async_copy(x_hbm.at[pl.ds(b * T + pos_base, PW)],
                         idxall.at[b], isem)
    for b in range(B):
        pltpu.make_async_copy(x_hbm.at[pl.ds(b * T + pos_base, PW)],
                              idxall.at[b], isem).wait()
    start_pos(0)
    apA0 = prep(0, 0)
    start_gather(0)
    apA1 = prep(1, 1)
    start_gather(1)

    def body(s, carry):
        a0, a1 = carry
        it0 = 4 * s

        wait_pos()
        wait_gather(0)
        wait_gather(1)

        @pl.when(s > 0)
        def _():
            wait_out(2)
            wait_out(3)

        a2 = prep(it0 + 2, 2)
        a3 = prep(it0 + 3, 3)
        start_gather(2)
        start_gather(3)

        fused_compute(0, 1, a0, a1)
        start_out(it0, 0)
        start_out(it0 + 1, 1)

        an0 = prep(jnp.minimum(it0 + 4, NIT - 1), 0)
        an1 = prep(jnp.minimum(it0 + 5, NIT - 1), 1)

        wait_gather(2)
        wait_gather(3)
        fused_compute(2, 3, a2, a3)
        start_out(it0 + 2, 2)
        start_out(it0 + 3, 3)

        @pl.when(s < NCH - 1)
        def _():
            wait_out(0)
            wait_out(1)
            start_gather(0)
            start_gather(1)
            start_pos(s + 1)

        return (an0, an1)

    lax.fori_loop(0, NCH, body, (apA0, apA1))

    wait_out(0)
    wait_out(1)
    wait_out(2)
    wait_out(3)


def kernel(x, emb_table, pos_table):
    out = _emb_lookup(x.reshape(-1).astype(jnp.int32), emb_table, pos_table)
    return out.reshape(B, T, D)


# speedup vs baseline: 1.1929x; 1.0661x over previous
import functools

import jax
import jax.numpy as jnp
from jax import lax
from jax.experimental import pallas as pl
from jax.experimental.pallas import tpu as pltpu
from jax.experimental.pallas import tpu_sc as plsc

B, T, D = 4, 8192, 768
PAD = 100000
NC, NS = 2, 16
NW = NC * NS
PW = T // NW
C = 32
NCH = PW // C
KV = D // 16
NIT = NCH * B

_DN = lax.GatherDimensionNumbers(
    offset_dims=(), collapsed_slice_dims=(0,), start_index_map=(0,))

_mesh = plsc.VectorSubcoreMesh(core_axis_name="c", subcore_axis_name="s")


@functools.partial(
    pl.kernel,
    out_type=jax.ShapeDtypeStruct((B * T, D), jnp.float32),
    mesh=_mesh,
    scratch_types=[
        pltpu.VMEM((B, PW), jnp.int32),
        pltpu.VMEM((4, C), jnp.int32),
        pltpu.VMEM((4, C), jnp.float32),
        pltpu.VMEM((C, D), jnp.float32),
        pltpu.VMEM((4, C, D), jnp.float32),
        pltpu.SemaphoreType.DMA,
        pltpu.SemaphoreType.DMA((4,)),
        pltpu.SemaphoreType.DMA((4,)),
        pltpu.SemaphoreType.DMA,
    ],
)
def _emb_lookup(x_hbm, emb_hbm, pos_hbm, out_hbm,
                idxall, idxs4, mask4, pbuf, ebuf, isem, gsem, osem, psem):
    wid = lax.axis_index("s") * NC + lax.axis_index("c")
    pos_base = wid * PW

    def prep(it, slot):
        b, pc = it % B, it // B
        padv = jnp.zeros((16,), jnp.int32)
        for k in range(C // 16):
            sl = pl.ds(k * 16, 16)
            v = idxall[b, pl.ds(pc * C + k * 16, 16)]
            ispad = v == PAD
            idxs4[slot, sl] = jnp.where(ispad, 0, v)
            mask4[slot, sl] = jnp.where(ispad, 1.0, 0.0)
            padv = padv | jnp.where(ispad, 1, 0)
        lanes = lax.iota(jnp.int32, 16)
        for sh in (8, 4, 2, 1):
            perm = ((lanes + sh) & 15)[:, None]
            padv = padv | lax.gather(
                padv, perm, _DN, (1,),
                mode=lax.GatherScatterMode.PROMISE_IN_BOUNDS)
        return padv[0]

    def start_gather(slot):
        pltpu.async_copy(emb_hbm.at[idxs4.at[slot]], ebuf.at[slot],
                         gsem.at[slot])

    def wait_gather(slot):
        pltpu.make_async_copy(emb_hbm.at[idxs4.at[slot]], ebuf.at[slot],
                              gsem.at[slot]).wait()

    def start_pos(pc):
        pltpu.async_copy(pos_hbm.at[pl.ds(pos_base + pc * C, C)], pbuf, psem)

    def wait_pos():
        pltpu.make_async_copy(pos_hbm.at[pl.ds(0, C)], pbuf, psem).wait()

    def start_out(it, slot):
        flat0 = (it % B) * T + pos_base + (it // B) * C
        pltpu.async_copy(ebuf.at[slot], out_hbm.at[pl.ds(flat0, C)],
                         osem.at[slot])

    def wait_out(slot):
        pltpu.make_async_copy(ebuf.at[slot], out_hbm.at[pl.ds(0, C)],
                              osem.at[slot]).wait()

    def rare_fix(slot, anypad):
        @pl.when(anypad != 0)
        def _():
            def rrow(r, c2):
                g16 = pl.multiple_of((r // 16) * 16, 16)
                mv = mask4[slot, pl.ds(g16, 16)]
                m = lax.gather(
                    mv, jnp.full((16, 1), r % 16, jnp.int32), _DN, (1,),
                    mode=lax.GatherScatterMode.PROMISE_IN_BOUNDS)
                km = 1.0 - m

                def rk(k, c3):
                    sl = pl.ds(pl.multiple_of(k * 16, 16), 16)
                    ebuf[slot, r, sl] = (km * ebuf[slot, r, sl]
                                         + m * pbuf[r, sl])
                    return c3

                lax.fori_loop(0, KV, rk, 0)
                return c2

            lax.fori_loop(0, C, rrow, 0)

    def fused_compute(slotA, slotB, apA, apB):
        @plsc.parallel_loop(0, C, 1)
        def crow(r):
            for k in range(KV):
                sl = pl.ds(k * 16, 16)
                v = pbuf[r, sl]
                plsc.addupdate(ebuf.at[slotA, r, sl], v)
                plsc.addupdate(ebuf.at[slotB, r, sl], v)

        rare_fix(slotA, apA)
        rare_fix(slotB, apB)

    for b in range(B):
        pltpu.async_copy(x_hbm.at[pl.ds(b * T + pos_base, PW)],
                         idxall.at[b], isem)
    for b in range(B):
        pltpu.make_async_copy(x_hbm.at[pl.ds(b * T + pos_base, PW)],
                              idxall.at[b], isem).wait()
    start_pos(0)
    apA0 = prep(0, 0)
    start_gather(0)
    apA1 = prep(1, 1)
    start_gather(1)

    def body(s, carry):
        a0, a1 = carry
        it0 = 4 * s

        wait_pos()
        wait_gather(0)
        wait_gather(1)

        @pl.when(s > 0)
        def _():
            wait_out(2)
            wait_out(3)

        a2 = prep(it0 + 2, 2)
        a3 = prep(it0 + 3, 3)
        start_gather(2)
        start_gather(3)

        fused_compute(0, 1, a0, a1)
        start_out(it0, 0)
        start_out(it0 + 1, 1)

        an0 = prep(jnp.minimum(it0 + 4, NIT - 1), 0)
        an1 = prep(jnp.minimum(it0 + 5, NIT - 1), 1)

        @pl.when(s < NCH - 1)
        def _():
            wait_out(0)
            wait_out(1)
            start_gather(0)
            start_gather(1)

        wait_gather(2)
        wait_gather(3)
        fused_compute(2, 3, a2, a3)
        start_out(it0 + 2, 2)
        start_out(it0 + 3, 3)

        @pl.when(s < NCH - 1)
        def _():
            start_pos(s + 1)

        return (an0, an1)

    lax.fori_loop(0, NCH, body, (apA0, apA1))

    wait_out(0)
    wait_out(1)
    wait_out(2)
    wait_out(3)


def kernel(x, emb_table, pos_table):
    out = _emb_lookup(x.reshape(-1).astype(jnp.int32), emb_table, pos_table)
    return out.reshape(B, T, D)
